# baseline (speedup 1.0000x reference)
import jax
import jax.numpy as jnp
from jax.experimental import pallas as pl
from jax.experimental.pallas import tpu as pltpu

W_POINT, W_CHAMFER, W_STRUCTURE = 1.0, 5.0, 2.0
BIG = 1e10


def _split3_bf16(x):
    hi = x.astype(jnp.bfloat16)
    r1 = x - hi.astype(jnp.float32)
    mid = r1.astype(jnp.bfloat16)
    lo = (r1 - mid.astype(jnp.float32)).astype(jnp.bfloat16)
    return hi, mid, lo


def _loss_kernel(rows, out_total, out_point, out_chamfer,
                 buf_a, buf_b, buf_c, buf_d):
    B = rows.shape[0]
    N = rows.shape[2]
    bf16 = jnp.bfloat16
    f32 = jnp.float32
    one = jnp.ones((1, N), bf16)
    dn = (((0,), (0,)), ((), ()))

    pxa, pya = rows[:, 0, :], rows[:, 1, :]
    txa, tya, tva = rows[:, 3, :], rows[:, 4, :], rows[:, 5, :]
    s0a, s1a = rows[:, 6, :], rows[:, 7, :]
    va = (tva == 1.0).astype(f32)
    exa = pxa - txa
    eya = pya - tya
    err2 = exa * exa + eya * eya
    pt_acc = jnp.sum(va * err2)
    tmaska = jnp.clip(s0a + s1a, 0.0, 1.0) * va
    st_acc = jnp.sum(tmaska * err2)
    tm_acc = jnp.sum(tmaska)

    def fill(b, buf):
        blk = rows[b]
        px, py, pv = blk[0:1], blk[1:2], blk[2:3]
        tx, ty, tv = blk[3:4], blk[4:5], blk[5:6]
        pm = (pv == 1.0).astype(f32)
        v = (tv == 1.0).astype(f32)
        a2p = px * px + py * py + (1.0 - pm) * BIG
        b2t = tx * tx + ty * ty + (1.0 - v) * BIG
        ahi, amid, alo = _split3_bf16(a2p)
        bhi, bmid, blo = _split3_bf16(b2t)
        p_pen = jnp.concatenate(
            [-2.0 * px.astype(bf16), -2.0 * py.astype(bf16),
             ahi, amid, alo, one, one, one], axis=0)
        t_pen = jnp.concatenate(
            [tx.astype(bf16), ty.astype(bf16), one, one, one,
             bhi, bmid, blo], axis=0)
        buf[...] = jax.lax.dot_general(p_pen, t_pen, dn,
                                       preferred_element_type=f32)

    def reduce(b, buf):
        blk = rows[b]
        pv, tv = blk[2:3], blk[5:6]
        v = (tv == 1.0).astype(f32)
        pm = (pv == 1.0).astype(f32)
        cnt_p = jnp.sum(pm)
        cnt_t = jnp.sum(v)
        pen_p = (1.0 - pm) * BIG
        pen_t = (1.0 - v) * BIG

        m = buf[...]
        q = N // 4
        s = jnp.minimum(jnp.minimum(m[0 * q:1 * q], m[1 * q:2 * q]),
                        jnp.minimum(m[2 * q:3 * q], m[3 * q:4 * q]))
        tgtmin = jnp.min(s, axis=0, keepdims=True)
        d2t = jnp.maximum(tgtmin - pen_t, 0.0) + 1e-12
        mean_t = jnp.sum(v * jnp.sqrt(d2t)) / jnp.maximum(cnt_t, 1.0)

        w = jnp.minimum(jnp.minimum(m[:, 0 * q:1 * q], m[:, 1 * q:2 * q]),
                        jnp.minimum(m[:, 2 * q:3 * q], m[:, 3 * q:4 * q]))
        z = jnp.min(w, axis=1, keepdims=True)
        d2p = jnp.maximum(jnp.transpose(z, (1, 0)) - pen_p, 0.0) + 1e-12
        mean_p = jnp.sum(pm * jnp.sqrt(d2p)) / jnp.maximum(cnt_p, 1.0)

        valid_b = ((cnt_p > 0.0) & (cnt_t > 0.0)).astype(f32)
        return valid_b * (mean_p + mean_t) * 0.5

    bufs = (buf_a, buf_b, buf_c, buf_d)
    fill(0, buf_a)
    fill(1, buf_b)

    def body(k, ch):
        s = 4 * k
        for c in range(4):
            fill(jnp.minimum(s + c + 2, B - 1), bufs[(c + 2) % 4])
            ch = ch + reduce(s + c, bufs[c])
        return ch

    ch_acc = jax.lax.fori_loop(0, B // 4, body, jnp.float32(0.0))

    n_elems = jnp.float32(B) * jnp.float32(2 * N)
    loss_point = pt_acc / n_elems
    loss_structure = jnp.where(tm_acc == 0.0, 0.0, st_acc / n_elems)
    loss_chamfer = ch_acc / jnp.float32(B)
    out_point[0, 0] = loss_point
    out_chamfer[0, 0] = loss_chamfer
    out_total[0, 0] = (W_POINT * loss_point + W_CHAMFER * loss_chamfer
                       + W_STRUCTURE * loss_structure)


def kernel(pred, target, skeleton_mask):
    B, N, _ = pred.shape
    f32 = jnp.float32

    rows = jnp.stack(
        [pred[:, :, 0], pred[:, :, 1], pred[:, :, 2],
         target[:, :, 0], target[:, :, 1], target[:, :, 2],
         skeleton_mask[:, :, 0].astype(f32),
         skeleton_mask[:, :, 1].astype(f32)], axis=1)

    out_spec = pl.BlockSpec(memory_space=pltpu.SMEM)
    out_shape = [jax.ShapeDtypeStruct((1, 1), f32)] * 3
    total, point, chamfer = pl.pallas_call(
        _loss_kernel,
        out_specs=[out_spec, out_spec, out_spec],
        out_shape=out_shape,
        scratch_shapes=[pltpu.VMEM((N, N), f32)] * 4,
    )(rows)

    return (total[0, 0], point[0, 0], jnp.zeros((), f32), chamfer[0, 0])


# speedup vs baseline: 1.0926x; 1.0926x over previous
import jax
import jax.numpy as jnp
from jax.experimental import pallas as pl
from jax.experimental.pallas import tpu as pltpu

W_POINT, W_CHAMFER, W_STRUCTURE = 1.0, 5.0, 2.0
BIG = 1e10


def _split3_bf16(x):
    hi = x.astype(jnp.bfloat16)
    r1 = x - hi.astype(jnp.float32)
    mid = r1.astype(jnp.bfloat16)
    lo = (r1 - mid.astype(jnp.float32)).astype(jnp.bfloat16)
    return hi, mid, lo


def _loss_kernel(rows, out_total, out_point, out_chamfer,
                 buf_a, buf_b, buf_c, buf_d):
    B = rows.shape[0]
    N = rows.shape[2]
    bf16 = jnp.bfloat16
    f32 = jnp.float32
    one = jnp.ones((1, N), bf16)
    dn = (((0,), (0,)), ((), ()))

    pxa, pya = rows[:, 0, :], rows[:, 1, :]
    txa, tya, tva = rows[:, 3, :], rows[:, 4, :], rows[:, 5, :]
    s0a, s1a = rows[:, 6, :], rows[:, 7, :]
    va = (tva == 1.0).astype(f32)
    exa = pxa - txa
    eya = pya - tya
    err2 = exa * exa + eya * eya
    pt_acc = jnp.sum(va * err2)
    tmaska = jnp.clip(s0a + s1a, 0.0, 1.0) * va
    st_acc = jnp.sum(tmaska * err2)
    tm_acc = jnp.sum(tmaska)

    def fill(b, buf):
        blk = rows[b]
        px, py, pv = blk[0:1], blk[1:2], blk[2:3]
        tx, ty, tv = blk[3:4], blk[4:5], blk[5:6]
        pm = (pv == 1.0).astype(f32)
        v = (tv == 1.0).astype(f32)
        a2p = px * px + py * py + (1.0 - pm) * BIG
        b2t = tx * tx + ty * ty + (1.0 - v) * BIG
        ahi, amid, alo = _split3_bf16(a2p)
        bhi, bmid, blo = _split3_bf16(b2t)
        p_pen = jnp.concatenate(
            [-2.0 * px.astype(bf16), -2.0 * py.astype(bf16),
             ahi, amid, alo, one, one, one], axis=0)
        t_pen = jnp.concatenate(
            [tx.astype(bf16), ty.astype(bf16), one, one, one,
             bhi, bmid, blo], axis=0)
        buf[...] = jax.lax.dot_general(p_pen, t_pen, dn,
                                       preferred_element_type=f32)

    def reduce(b, buf):
        blk = rows[b]
        pv, tv = blk[2:3], blk[5:6]
        v = (tv == 1.0).astype(f32)
        pm = (pv == 1.0).astype(f32)
        cnt_p = jnp.sum(pm)
        cnt_t = jnp.sum(v)
        pen_p = (1.0 - pm) * BIG
        pen_t = (1.0 - v) * BIG

        m = buf[...]
        tgtmin = jnp.min(m, axis=0, keepdims=True)
        d2t = jnp.maximum(tgtmin - pen_t, 0.0) + 1e-12
        mean_t = jnp.sum(v * jnp.sqrt(d2t)) / jnp.maximum(cnt_t, 1.0)

        z = jnp.min(m, axis=1, keepdims=True)
        d2p = jnp.maximum(jnp.transpose(z, (1, 0)) - pen_p, 0.0) + 1e-12
        mean_p = jnp.sum(pm * jnp.sqrt(d2p)) / jnp.maximum(cnt_p, 1.0)

        valid_b = ((cnt_p > 0.0) & (cnt_t > 0.0)).astype(f32)
        return valid_b * (mean_p + mean_t) * 0.5

    bufs = (buf_a, buf_b, buf_c, buf_d)
    fill(0, buf_a)

    def body(k, ch):
        s = 8 * k
        for c in range(8):
            fill(jnp.minimum(s + c + 1, B - 1), bufs[(c + 1) % 4])
            ch = ch + reduce(s + c, bufs[c % 4])
        return ch

    ch_acc = jax.lax.fori_loop(0, B // 8, body, jnp.float32(0.0))

    n_elems = jnp.float32(B) * jnp.float32(2 * N)
    loss_point = pt_acc / n_elems
    loss_structure = jnp.where(tm_acc == 0.0, 0.0, st_acc / n_elems)
    loss_chamfer = ch_acc / jnp.float32(B)
    out_point[0, 0] = loss_point
    out_chamfer[0, 0] = loss_chamfer
    out_total[0, 0] = (W_POINT * loss_point + W_CHAMFER * loss_chamfer
                       + W_STRUCTURE * loss_structure)


def kernel(pred, target, skeleton_mask):
    B, N, _ = pred.shape
    f32 = jnp.float32

    rows = jnp.stack(
        [pred[:, :, 0], pred[:, :, 1], pred[:, :, 2],
         target[:, :, 0], target[:, :, 1], target[:, :, 2],
         skeleton_mask[:, :, 0].astype(f32),
         skeleton_mask[:, :, 1].astype(f32)], axis=1)

    out_spec = pl.BlockSpec(memory_space=pltpu.SMEM)
    out_shape = [jax.ShapeDtypeStruct((1, 1), f32)] * 3
    total, point, chamfer = pl.pallas_call(
        _loss_kernel,
        out_specs=[out_spec, out_spec, out_spec],
        out_shape=out_shape,
        scratch_shapes=[pltpu.VMEM((N, N), f32)] * 4,
    )(rows)

    return (total[0, 0], point[0, 0], jnp.zeros((), f32), chamfer[0, 0])


# speedup vs baseline: 1.1198x; 1.0249x over previous
import jax
import jax.numpy as jnp
from jax.experimental import pallas as pl
from jax.experimental.pallas import tpu as pltpu

W_POINT, W_CHAMFER, W_STRUCTURE = 1.0, 5.0, 2.0
BIG = 1e10


def _split3_bf16(x):
    hi = x.astype(jnp.bfloat16)
    r1 = x - hi.astype(jnp.float32)
    mid = r1.astype(jnp.bfloat16)
    lo = (r1 - mid.astype(jnp.float32)).astype(jnp.bfloat16)
    return hi, mid, lo


def _loss_kernel(rows, out_total, out_point, out_chamfer,
                 buf_a, buf_b, buf_c, buf_d):
    B = rows.shape[0]
    N = rows.shape[2]
    bf16 = jnp.bfloat16
    f32 = jnp.float32
    one = jnp.ones((1, N), bf16)
    dn = (((0,), (0,)), ((), ()))

    pxa, pya = rows[:, 0, :], rows[:, 1, :]
    txa, tya, tva = rows[:, 3, :], rows[:, 4, :], rows[:, 5, :]
    s0a, s1a = rows[:, 6, :], rows[:, 7, :]
    va = (tva == 1.0).astype(f32)
    exa = pxa - txa
    eya = pya - tya
    err2 = exa * exa + eya * eya
    pt_acc = jnp.sum(va * err2)
    tmaska = jnp.clip(s0a + s1a, 0.0, 1.0) * va
    st_acc = jnp.sum(tmaska * err2)
    tm_acc = jnp.sum(tmaska)

    def fill(b, buf):
        blk = rows[b]
        px, py, pv = blk[0:1], blk[1:2], blk[2:3]
        tx, ty, tv = blk[3:4], blk[4:5], blk[5:6]
        pm = (pv == 1.0).astype(f32)
        v = (tv == 1.0).astype(f32)
        a2p = px * px + py * py + (1.0 - pm) * BIG
        b2t = tx * tx + ty * ty + (1.0 - v) * BIG
        ahi, amid, alo = _split3_bf16(a2p)
        bhi, bmid, blo = _split3_bf16(b2t)
        p_pen = jnp.concatenate(
            [-2.0 * px.astype(bf16), -2.0 * py.astype(bf16),
             ahi, amid, alo, one, one, one], axis=0)
        t_pen = jnp.concatenate(
            [tx.astype(bf16), ty.astype(bf16), one, one, one,
             bhi, bmid, blo], axis=0)
        buf[...] = jax.lax.dot_general(p_pen, t_pen, dn,
                                       preferred_element_type=f32)

    def reduce(b, buf):
        blk = rows[b]
        pv, tv = blk[2:3], blk[5:6]
        v = (tv == 1.0).astype(f32)
        pm = (pv == 1.0).astype(f32)
        cnt_p = jnp.sum(pm)
        cnt_t = jnp.sum(v)
        pen_p = (1.0 - pm) * BIG
        pen_t = (1.0 - v) * BIG

        m = buf[...]
        tgtmin = jnp.min(m, axis=0, keepdims=True)
        d2t = jnp.maximum(tgtmin - pen_t, 0.0) + 1e-12
        mean_t = jnp.sum(v * jnp.sqrt(d2t)) / jnp.maximum(cnt_t, 1.0)

        z = jnp.min(m, axis=1, keepdims=True)
        d2p = jnp.maximum(jnp.transpose(z, (1, 0)) - pen_p, 0.0) + 1e-12
        mean_p = jnp.sum(pm * jnp.sqrt(d2p)) / jnp.maximum(cnt_p, 1.0)

        valid_b = ((cnt_p > 0.0) & (cnt_t > 0.0)).astype(f32)
        return valid_b * (mean_p + mean_t) * 0.5

    bufs = (buf_a, buf_b, buf_c, buf_d)
    fill(0, buf_a)

    ch_acc = jnp.float32(0.0)
    for c in range(B):
        if c + 1 < B:
            fill(c + 1, bufs[(c + 1) % 4])
        ch_acc = ch_acc + reduce(c, bufs[c % 4])

    n_elems = jnp.float32(B) * jnp.float32(2 * N)
    loss_point = pt_acc / n_elems
    loss_structure = jnp.where(tm_acc == 0.0, 0.0, st_acc / n_elems)
    loss_chamfer = ch_acc / jnp.float32(B)
    out_point[0, 0] = loss_point
    out_chamfer[0, 0] = loss_chamfer
    out_total[0, 0] = (W_POINT * loss_point + W_CHAMFER * loss_chamfer
                       + W_STRUCTURE * loss_structure)


def kernel(pred, target, skeleton_mask):
    B, N, _ = pred.shape
    f32 = jnp.float32

    rows = jnp.stack(
        [pred[:, :, 0], pred[:, :, 1], pred[:, :, 2],
         target[:, :, 0], target[:, :, 1], target[:, :, 2],
         skeleton_mask[:, :, 0].astype(f32),
         skeleton_mask[:, :, 1].astype(f32)], axis=1)

    out_spec = pl.BlockSpec(memory_space=pltpu.SMEM)
    out_shape = [jax.ShapeDtypeStruct((1, 1), f32)] * 3
    total, point, chamfer = pl.pallas_call(
        _loss_kernel,
        out_specs=[out_spec, out_spec, out_spec],
        out_shape=out_shape,
        scratch_shapes=[pltpu.VMEM((N, N), f32)] * 4,
    )(rows)

    return (total[0, 0], point[0, 0], jnp.zeros((), f32), chamfer[0, 0])
